# baseline (speedup 1.0000x reference)
import functools

import jax
import jax.numpy as jnp
from jax import lax
from jax.experimental import pallas as pl
from jax.experimental.pallas import tpu as pltpu
from jax.experimental.pallas import tpu_sc as plsc

NUM_EMBEDDINGS = 100000
EMBEDDING_DIM = 128

B, L = 4096, 50
TOTAL = B * L
NC, NS = 2, 16
NW = NC * NS
PER_W = TOTAL // NW
CHUNK = 128
NCHUNK = PER_W // CHUNK
NBUF = 7
PF = 5

_STEADY_END = ((NCHUNK - PF - NBUF) // NBUF) * NBUF


def _sc_gather(idx_flat, weight):
    mesh = plsc.VectorSubcoreMesh(
        core_axis_name="c", subcore_axis_name="s", num_cores=NC, num_subcores=NS
    )

    @functools.partial(
        pl.kernel,
        out_type=jax.ShapeDtypeStruct((TOTAL, EMBEDDING_DIM), jnp.float32),
        mesh=mesh,
        scratch_types=[
            pltpu.VMEM((PER_W,), jnp.int32),
            pltpu.VMEM((NBUF, CHUNK, EMBEDDING_DIM), jnp.float32),
            pltpu.SemaphoreType.DMA,
            pltpu.SemaphoreType.DMA,
            pltpu.SemaphoreType.DMA,
        ],
    )
    def k(idx_hbm, table_hbm, out_hbm, idx_v, rows_v, sem_in, sem_out, sem_idx):
        wid = lax.axis_index("s") * NC + lax.axis_index("c")
        base = wid * PER_W
        head_idx = PF * CHUNK
        pltpu.sync_copy(
            idx_hbm.at[pl.ds(base, head_idx)], idx_v.at[pl.ds(0, head_idx)]
        )
        rest = pltpu.async_copy(
            idx_hbm.at[pl.ds(base + head_idx, PER_W - head_idx)],
            idx_v.at[pl.ds(head_idx, PER_W - head_idx)],
            sem_idx,
        )

        def start_gather(j, b):
            pltpu.async_copy(
                table_hbm.at[idx_v.at[pl.ds(j * CHUNK, CHUNK)]], rows_v.at[b], sem_in
            )

        def wait_gather(b):
            pltpu.make_async_copy(
                table_hbm.at[idx_v.at[pl.ds(0, CHUNK)]], rows_v.at[b], sem_in
            ).wait()

        def start_out(j, b):
            pltpu.async_copy(
                rows_v.at[b], out_hbm.at[pl.ds(base + j * CHUNK, CHUNK)], sem_out
            )

        def wait_out(b):
            pltpu.make_async_copy(
                rows_v.at[b], out_hbm.at[pl.ds(base, CHUNK)], sem_out
            ).wait()

        def step(j, b, do_wait_out, do_gather):
            wait_gather(b)
            start_out(j, b)
            if do_wait_out:
                wait_out((b + PF) % NBUF)
            if do_gather:
                start_gather(j + PF, (b + PF) % NBUF)

        for j in range(PF):
            start_gather(j, j)
        rest.wait()

        for j in range(NBUF):
            step(j, j, j >= NBUF - PF, True)

        @pl.loop(NBUF, NBUF + _STEADY_END, step=NBUF)
        def _(g):
            for b in range(NBUF):
                step(g + b, b, True, True)

        for j in range(NBUF + _STEADY_END, NCHUNK):
            step(j, j % NBUF, j + PF < NCHUNK, j + PF < NCHUNK)

        for _ in range(NBUF):
            wait_out(0)

    return k(idx_flat, weight)


def kernel(token_ids, weight):
    idx = token_ids.astype(jnp.int32).T.reshape(TOTAL)
    out = _sc_gather(idx, weight)
    return out.reshape(L, B, EMBEDDING_DIM).transpose(1, 0, 2)


# speedup vs baseline: 1.0032x; 1.0032x over previous
import functools

import jax
import jax.numpy as jnp
from jax import lax
from jax.experimental import pallas as pl
from jax.experimental.pallas import tpu as pltpu
from jax.experimental.pallas import tpu_sc as plsc

NUM_EMBEDDINGS = 100000
EMBEDDING_DIM = 128

B, L = 4096, 50
TOTAL = B * L
NC, NS = 2, 16
NW = NC * NS
PER_W = TOTAL // NW
CHUNK = 128
NCHUNK = PER_W // CHUNK
NBUF = 7
PF = 4

_STEADY_END = ((NCHUNK - PF - NBUF) // NBUF) * NBUF


def _sc_gather(idx_flat, weight):
    mesh = plsc.VectorSubcoreMesh(
        core_axis_name="c", subcore_axis_name="s", num_cores=NC, num_subcores=NS
    )

    @functools.partial(
        pl.kernel,
        out_type=jax.ShapeDtypeStruct((TOTAL, EMBEDDING_DIM), jnp.float32),
        mesh=mesh,
        scratch_types=[
            pltpu.VMEM((PER_W,), jnp.int32),
            pltpu.VMEM((NBUF, CHUNK, EMBEDDING_DIM), jnp.float32),
            pltpu.SemaphoreType.DMA,
            pltpu.SemaphoreType.DMA,
            pltpu.SemaphoreType.DMA,
        ],
    )
    def k(idx_hbm, table_hbm, out_hbm, idx_v, rows_v, sem_in, sem_out, sem_idx):
        wid = lax.axis_index("s") * NC + lax.axis_index("c")
        base = wid * PER_W
        head_idx = PF * CHUNK
        pltpu.sync_copy(
            idx_hbm.at[pl.ds(base, head_idx)], idx_v.at[pl.ds(0, head_idx)]
        )
        rest = pltpu.async_copy(
            idx_hbm.at[pl.ds(base + head_idx, PER_W - head_idx)],
            idx_v.at[pl.ds(head_idx, PER_W - head_idx)],
            sem_idx,
        )

        def start_gather(j, b):
            pltpu.async_copy(
                table_hbm.at[idx_v.at[pl.ds(j * CHUNK, CHUNK)]], rows_v.at[b], sem_in
            )

        def wait_gather(b):
            pltpu.make_async_copy(
                table_hbm.at[idx_v.at[pl.ds(0, CHUNK)]], rows_v.at[b], sem_in
            ).wait()

        def start_out(j, b):
            pltpu.async_copy(
                rows_v.at[b], out_hbm.at[pl.ds(base + j * CHUNK, CHUNK)], sem_out
            )

        def wait_out(b):
            pltpu.make_async_copy(
                rows_v.at[b], out_hbm.at[pl.ds(base, CHUNK)], sem_out
            ).wait()

        def step(j, b, do_wait_out, do_gather):
            wait_gather(b)
            start_out(j, b)
            if do_wait_out:
                wait_out((b + PF) % NBUF)
            if do_gather:
                start_gather(j + PF, (b + PF) % NBUF)

        for j in range(PF):
            start_gather(j, j)
        rest.wait()

        for j in range(NBUF):
            step(j, j, j >= NBUF - PF, True)

        @pl.loop(NBUF, NBUF + _STEADY_END, step=NBUF)
        def _(g):
            for b in range(NBUF):
                step(g + b, b, True, True)

        for j in range(NBUF + _STEADY_END, NCHUNK):
            step(j, j % NBUF, j + PF < NCHUNK, j + PF < NCHUNK)

        for _ in range(NBUF):
            wait_out(0)

    return k(idx_flat, weight)


def kernel(token_ids, weight):
    idx = token_ids.astype(jnp.int32).T.reshape(TOTAL)
    out = _sc_gather(idx, weight)
    return out.reshape(L, B, EMBEDDING_DIM).transpose(1, 0, 2)
